# baseline (speedup 1.0000x reference)
import functools

import numpy as np
import jax
import jax.numpy as jnp
from jax import lax
from jax.experimental import pallas as pl
from jax.experimental.pallas import tpu as pltpu
from jax.experimental.pallas import tpu_sc as plsc

N_NODES = 10000
N_EDGES = 160000
D_IN = 128
D_OUT = 15

NC = 2
NS = 16
NW = NC * NS
CHUNK = 125
NSLICE = 1
ES = N_EDGES // NSLICE
EW = ES // NW
CH = EW // CHUNK
RPT = N_NODES // NS
BE = 3200
QB = BE // 8

_q = np.arange(ES)
_EDGE_OF_SLOT = ((_q // 8 // QB) * BE + (_q % 8) * QB + (_q // 8) % QB)

_U_SEL = [0, 1, 2, 3, 4, 4, 4, 5, 5, 5, 6, 6, 6, 6, 6]
_S_SEL = [0, 0, 0, 0, 1, 2, 3, 1, 2, 3, 4, 5, 6, 7, 8]


def _sc_gather(nf, src2d):
    mesh = plsc.VectorSubcoreMesh(core_axis_name="c", subcore_axis_name="s")

    @functools.partial(
        pl.kernel,
        mesh=mesh,
        out_type=jax.ShapeDtypeStruct((ES, D_IN), jnp.float32),
        scratch_types=[
            pltpu.VMEM((CH, CHUNK), jnp.int32),
            pltpu.VMEM((CHUNK, D_IN), jnp.float32),
            pltpu.VMEM((CHUNK, D_IN), jnp.float32),
            pltpu.SemaphoreType.DMA,
            pltpu.SemaphoreType.DMA,
        ],
        compiler_params=pltpu.CompilerParams(use_tc_tiling_on_sc=False),
    )
    def k(nf_hbm, src_hbm, out_hbm, idx_v, buf0, buf1, sem0, sem1):
        c = lax.axis_index("c")
        s = lax.axis_index("s")
        wid = s * NC + c
        pltpu.sync_copy(src_hbm.at[pl.ds(wid * CH, CH)], idx_v)
        bufs = (buf0, buf1)
        sems = (sem0, sem1)
        pltpu.async_copy(nf_hbm.at[idx_v.at[0]], buf0, sem0)

        def body(j, carry):
            slot = lax.rem(j, 2)

            def step(b, sm, other_b, other_sm):
                @pl.when(j + 1 < CH)
                def _start():
                    pltpu.async_copy(nf_hbm.at[idx_v.at[j + 1]], other_b, other_sm)

                pltpu.make_async_copy(nf_hbm.at[idx_v.at[j]], b, sm).wait()
                pltpu.sync_copy(b, out_hbm.at[pl.ds(wid * EW + j * CHUNK, CHUNK)])

            @pl.when(slot == 0)
            def _even():
                step(bufs[0], sems[0], bufs[1], sems[1])

            @pl.when(slot == 1)
            def _odd():
                step(bufs[1], sems[1], bufs[0], sems[0])

            return carry

        lax.fori_loop(0, CH, body, 0)

    return k(nf, src2d)


def _sc_scatter(msg, dst2d, zmat):
    mesh = plsc.VectorSubcoreMesh(core_axis_name="c", subcore_axis_name="s")

    @functools.partial(
        pl.kernel,
        mesh=mesh,
        out_type=jax.ShapeDtypeStruct((NC, N_NODES, 16), jnp.float32),
        scratch_types=[
            pltpu.VMEM((CH, CHUNK), jnp.int32),
            pltpu.VMEM((EW, 16), jnp.float32),
            pltpu.VMEM((RPT, 16), jnp.float32),
            pltpu.VMEM_SHARED((N_NODES, 16), jnp.float32),
        ],
        compiler_params=pltpu.CompilerParams(use_tc_tiling_on_sc=False),
    )
    def k(msg_hbm, dst_hbm, z_hbm, out_hbm, idx_v, msg_v, bnc, acc):
        c = lax.axis_index("c")
        s = lax.axis_index("s")
        wid = s * NC + c
        pltpu.sync_copy(z_hbm.at[pl.ds(s * RPT, RPT)], bnc)
        pltpu.sync_copy(bnc, acc.at[pl.ds(s * RPT, RPT)])
        pltpu.sync_copy(dst_hbm.at[pl.ds(wid * CH, CH)], idx_v)
        pltpu.sync_copy(msg_hbm.at[pl.ds(wid * EW, EW)], msg_v)
        plsc.subcore_barrier()

        def body(j, carry):
            pltpu.sync_copy(
                msg_v.at[pl.ds(j * CHUNK, CHUNK)], acc.at[idx_v.at[j]], add=True
            )
            return carry

        lax.fori_loop(0, CH, body, 0)
        plsc.subcore_barrier()
        pltpu.sync_copy(acc.at[pl.ds(s * RPT, RPT)], bnc)
        pltpu.sync_copy(bnc, out_hbm.at[c, pl.ds(s * RPT, RPT)])

    return k(msg, dst2d, zmat)


def _tc_dense(gathered, radial, shp, W1s, W2q, A, B):
    def body(g_ref, r_ref, sh_ref, w1_ref, w2_ref, a_ref, b_ref, o_ref):
        r = r_ref[...]
        h1 = jnp.dot(r, w1_ref[...], preferred_element_type=jnp.float32)
        h = h1 / (1.0 + jnp.exp(-h1))
        w = jnp.dot(h.astype(jnp.bfloat16), w2_ref[...],
                    preferred_element_type=jnp.float32)
        g = g_ref[...]
        cols = []
        for u in range(7):
            cols.append(jnp.sum(w[:, u * D_IN:(u + 1) * D_IN] * g, axis=1,
                                keepdims=True))
        cols.append(jnp.zeros_like(cols[0]))
        proj = jnp.concatenate(cols, axis=1)
        pe = jnp.dot(proj, a_ref[...], preferred_element_type=jnp.float32)
        se = jnp.dot(sh_ref[...], b_ref[...], preferred_element_type=jnp.float32)
        msg = pe * se
        o_ref[...] = jnp.concatenate(
            [msg[k * QB:(k + 1) * QB] for k in range(8)], axis=1)

    return pl.pallas_call(
        body,
        grid=(ES // BE,),
        in_specs=[
            pl.BlockSpec((BE, D_IN), lambda i: (i, 0)),
            pl.BlockSpec((BE, 64), lambda i: (i, 0)),
            pl.BlockSpec((BE, 9), lambda i: (i, 0)),
            pl.BlockSpec((64, 64), lambda i: (0, 0)),
            pl.BlockSpec((64, 896), lambda i: (0, 0)),
            pl.BlockSpec((8, 16), lambda i: (0, 0)),
            pl.BlockSpec((9, 16), lambda i: (0, 0)),
        ],
        out_specs=pl.BlockSpec((QB, 128), lambda i: (i, 0)),
        out_shape=jax.ShapeDtypeStruct((ES // 8, 128), jnp.float32),
    )(gathered, radial, shp, W1s, W2q, A, B)


def _tc_final(partials, nf, wselfp):
    def body(p_ref, nf_ref, ws_ref, o_ref):
        s0 = jnp.dot(nf_ref[...], ws_ref[...], preferred_element_type=jnp.float32)
        acc = s0
        for t in range(NSLICE * NC):
            acc = acc + p_ref[t]
        o_ref[...] = acc

    return pl.pallas_call(
        body,
        out_shape=jax.ShapeDtypeStruct((N_NODES, 16), jnp.float32),
    )(partials, nf, wselfp)


def kernel(node_features, edge_index, edge_sh, edge_radial, W1, W2, W_self):
    src = edge_index[0]
    dst = edge_index[1]
    eq = jnp.asarray(_EDGE_OF_SLOT, dtype=jnp.int32)

    W1s = (W1 * (1.0 / np.sqrt(64.0))).astype(jnp.bfloat16)
    w2_scale = 1.0 / (np.sqrt(64.0) * np.sqrt(float(D_IN)) * 4.0)
    W2q = (W2.reshape(64, D_IN, 7).transpose(0, 2, 1).reshape(64, 7 * D_IN)
           * w2_scale).astype(jnp.bfloat16)

    A = np.zeros((8, 16), np.float32)
    B = np.zeros((16, 16), np.float32)
    for o in range(D_OUT):
        A[_U_SEL[o], o] = 1.0
        B[_S_SEL[o], o] = 1.0
    A = jnp.asarray(A)
    B9 = jnp.asarray(B[:9]).astype(jnp.bfloat16)

    wselfp = jnp.pad(W_self, ((0, 0), (0, 16 - 4))) * (1.0 / np.sqrt(float(D_IN)))
    zmat = jnp.zeros((N_NODES, 16), jnp.float32)

    radial_bf = edge_radial.astype(jnp.bfloat16)
    sh_bf = edge_sh.astype(jnp.bfloat16)

    partials = []
    for sl in range(NSLICE):
        lo = sl * ES
        src2d = src[lo:lo + ES].reshape(ES // CHUNK, CHUNK)
        dst2d = dst[lo:lo + ES][eq].reshape(ES // CHUNK, CHUNK)
        gathered = _sc_gather(node_features, src2d)
        msg8 = _tc_dense(gathered, radial_bf[lo:lo + ES], sh_bf[lo:lo + ES],
                         W1s, W2q, A, B9)
        partials.append(_sc_scatter(msg8.reshape(ES, 16), dst2d, zmat))

    pall = jnp.concatenate(partials, axis=0)
    out16 = _tc_final(pall, node_features, wselfp)
    return out16[:, :D_OUT]


# speedup vs baseline: 1.0178x; 1.0178x over previous
import functools

import numpy as np
import jax
import jax.numpy as jnp
from jax import lax
from jax.experimental import pallas as pl
from jax.experimental.pallas import tpu as pltpu
from jax.experimental.pallas import tpu_sc as plsc

N_NODES = 10000
N_EDGES = 160000
D_IN = 128
D_OUT = 15

NC = 2
NS = 16
NW = NC * NS
CHUNK = 125
NSLICE = 1
ES = N_EDGES // NSLICE
EW = ES // NW
CH = EW // CHUNK
RPT = N_NODES // NS
BE = 6400
QB = BE // 8

_q = np.arange(ES)
_EDGE_OF_SLOT = ((_q // 8 // QB) * BE + (_q % 8) * QB + (_q // 8) % QB)

_U_SEL = [0, 1, 2, 3, 4, 4, 4, 5, 5, 5, 6, 6, 6, 6, 6]
_S_SEL = [0, 0, 0, 0, 1, 2, 3, 1, 2, 3, 4, 5, 6, 7, 8]


def _sc_gather(nf, src2d):
    mesh = plsc.VectorSubcoreMesh(core_axis_name="c", subcore_axis_name="s")

    @functools.partial(
        pl.kernel,
        mesh=mesh,
        out_type=jax.ShapeDtypeStruct((ES, D_IN), jnp.float32),
        scratch_types=[
            pltpu.VMEM((CH, CHUNK), jnp.int32),
            pltpu.VMEM((CHUNK, D_IN), jnp.float32),
            pltpu.VMEM((CHUNK, D_IN), jnp.float32),
            pltpu.SemaphoreType.DMA,
            pltpu.SemaphoreType.DMA,
        ],
        compiler_params=pltpu.CompilerParams(use_tc_tiling_on_sc=False),
    )
    def k(nf_hbm, src_hbm, out_hbm, idx_v, buf0, buf1, sem0, sem1):
        c = lax.axis_index("c")
        s = lax.axis_index("s")
        wid = s * NC + c
        pltpu.sync_copy(src_hbm.at[pl.ds(wid * CH, CH)], idx_v)
        bufs = (buf0, buf1)
        sems = (sem0, sem1)
        pltpu.async_copy(nf_hbm.at[idx_v.at[0]], buf0, sem0)

        def body(j, carry):
            slot = lax.rem(j, 2)

            def step(b, sm, other_b, other_sm):
                @pl.when(j + 1 < CH)
                def _start():
                    pltpu.async_copy(nf_hbm.at[idx_v.at[j + 1]], other_b, other_sm)

                pltpu.make_async_copy(nf_hbm.at[idx_v.at[j]], b, sm).wait()
                pltpu.sync_copy(b, out_hbm.at[pl.ds(wid * EW + j * CHUNK, CHUNK)])

            @pl.when(slot == 0)
            def _even():
                step(bufs[0], sems[0], bufs[1], sems[1])

            @pl.when(slot == 1)
            def _odd():
                step(bufs[1], sems[1], bufs[0], sems[0])

            return carry

        lax.fori_loop(0, CH, body, 0)

    return k(nf, src2d)


def _sc_scatter(msg, dst2d, zmat):
    mesh = plsc.VectorSubcoreMesh(core_axis_name="c", subcore_axis_name="s")

    @functools.partial(
        pl.kernel,
        mesh=mesh,
        out_type=jax.ShapeDtypeStruct((NC, N_NODES, 16), jnp.float32),
        scratch_types=[
            pltpu.VMEM((CH, CHUNK), jnp.int32),
            pltpu.VMEM((EW, 16), jnp.float32),
            pltpu.VMEM((RPT, 16), jnp.float32),
            pltpu.VMEM_SHARED((N_NODES, 16), jnp.float32),
        ],
        compiler_params=pltpu.CompilerParams(use_tc_tiling_on_sc=False),
    )
    def k(msg_hbm, dst_hbm, z_hbm, out_hbm, idx_v, msg_v, bnc, acc):
        c = lax.axis_index("c")
        s = lax.axis_index("s")
        wid = s * NC + c
        pltpu.sync_copy(z_hbm.at[pl.ds(s * RPT, RPT)], bnc)
        pltpu.sync_copy(bnc, acc.at[pl.ds(s * RPT, RPT)])
        pltpu.sync_copy(dst_hbm.at[pl.ds(wid * CH, CH)], idx_v)
        pltpu.sync_copy(msg_hbm.at[pl.ds(wid * EW, EW)], msg_v)
        plsc.subcore_barrier()

        def body(j, carry):
            pltpu.sync_copy(
                msg_v.at[pl.ds(j * CHUNK, CHUNK)], acc.at[idx_v.at[j]], add=True
            )
            return carry

        lax.fori_loop(0, CH, body, 0)
        plsc.subcore_barrier()
        pltpu.sync_copy(acc.at[pl.ds(s * RPT, RPT)], bnc)
        pltpu.sync_copy(bnc, out_hbm.at[c, pl.ds(s * RPT, RPT)])

    return k(msg, dst2d, zmat)


def _tc_dense(gathered, radial, shp, W1s, W2q, A, B):
    def body(g_ref, r_ref, sh_ref, w1_ref, w2_ref, a_ref, b_ref, o_ref):
        r = r_ref[...]
        h1 = jnp.dot(r, w1_ref[...], preferred_element_type=jnp.float32)
        h = h1 / (1.0 + jnp.exp(-h1))
        w = jnp.dot(h.astype(jnp.bfloat16), w2_ref[...],
                    preferred_element_type=jnp.float32)
        g = g_ref[...]
        cols = []
        for u in range(7):
            cols.append(jnp.sum(w[:, u * D_IN:(u + 1) * D_IN] * g, axis=1,
                                keepdims=True))
        cols.append(jnp.zeros_like(cols[0]))
        proj = jnp.concatenate(cols, axis=1)
        pe = jnp.dot(proj, a_ref[...], preferred_element_type=jnp.float32)
        se = jnp.dot(sh_ref[...], b_ref[...], preferred_element_type=jnp.float32)
        msg = pe * se
        o_ref[...] = jnp.concatenate(
            [msg[k * QB:(k + 1) * QB] for k in range(8)], axis=1)

    return pl.pallas_call(
        body,
        grid=(ES // BE,),
        in_specs=[
            pl.BlockSpec((BE, D_IN), lambda i: (i, 0)),
            pl.BlockSpec((BE, 64), lambda i: (i, 0)),
            pl.BlockSpec((BE, 9), lambda i: (i, 0)),
            pl.BlockSpec((64, 64), lambda i: (0, 0)),
            pl.BlockSpec((64, 896), lambda i: (0, 0)),
            pl.BlockSpec((8, 16), lambda i: (0, 0)),
            pl.BlockSpec((9, 16), lambda i: (0, 0)),
        ],
        out_specs=pl.BlockSpec((QB, 128), lambda i: (i, 0)),
        out_shape=jax.ShapeDtypeStruct((ES // 8, 128), jnp.float32),
    )(gathered, radial, shp, W1s, W2q, A, B)


def _tc_final(partials, nf, wselfp):
    def body(p_ref, nf_ref, ws_ref, o_ref):
        s0 = jnp.dot(nf_ref[...], ws_ref[...], preferred_element_type=jnp.float32)
        acc = s0
        for t in range(NSLICE * NC):
            acc = acc + p_ref[t]
        o_ref[...] = acc

    return pl.pallas_call(
        body,
        out_shape=jax.ShapeDtypeStruct((N_NODES, 16), jnp.float32),
    )(partials, nf, wselfp)


def kernel(node_features, edge_index, edge_sh, edge_radial, W1, W2, W_self):
    src = edge_index[0]
    dst = edge_index[1]
    eq = jnp.asarray(_EDGE_OF_SLOT, dtype=jnp.int32)

    W1s = (W1 * (1.0 / np.sqrt(64.0))).astype(jnp.bfloat16)
    w2_scale = 1.0 / (np.sqrt(64.0) * np.sqrt(float(D_IN)) * 4.0)
    W2q = (W2.reshape(64, D_IN, 7).transpose(0, 2, 1).reshape(64, 7 * D_IN)
           * w2_scale).astype(jnp.bfloat16)

    A = np.zeros((8, 16), np.float32)
    B = np.zeros((16, 16), np.float32)
    for o in range(D_OUT):
        A[_U_SEL[o], o] = 1.0
        B[_S_SEL[o], o] = 1.0
    A = jnp.asarray(A)
    B9 = jnp.asarray(B[:9]).astype(jnp.bfloat16)

    wselfp = jnp.pad(W_self, ((0, 0), (0, 16 - 4))) * (1.0 / np.sqrt(float(D_IN)))
    zmat = jnp.zeros((N_NODES, 16), jnp.float32)

    radial_bf = edge_radial.astype(jnp.bfloat16)
    sh_bf = edge_sh.astype(jnp.bfloat16)

    partials = []
    for sl in range(NSLICE):
        lo = sl * ES
        src2d = src[lo:lo + ES].reshape(ES // CHUNK, CHUNK)
        dst2d = dst[lo:lo + ES][eq].reshape(ES // CHUNK, CHUNK)
        gathered = _sc_gather(node_features, src2d)
        msg8 = _tc_dense(gathered, radial_bf[lo:lo + ES], sh_bf[lo:lo + ES],
                         W1s, W2q, A, B9)
        partials.append(_sc_scatter(msg8.reshape(ES, 16), dst2d, zmat))

    pall = jnp.concatenate(partials, axis=0)
    out16 = _tc_final(pall, node_features, wselfp)
    return out16[:, :D_OUT]


# speedup vs baseline: 1.0194x; 1.0016x over previous
import functools

import numpy as np
import jax
import jax.numpy as jnp
from jax import lax
from jax.experimental import pallas as pl
from jax.experimental.pallas import tpu as pltpu
from jax.experimental.pallas import tpu_sc as plsc

N_NODES = 10000
N_EDGES = 160000
D_IN = 128
D_OUT = 15

NC = 2
NS = 16
NW = NC * NS
CHUNK = 125
NSLICE = 1
ES = N_EDGES // NSLICE
EW = ES // NW
CH = EW // CHUNK
RPT = N_NODES // NS
BE = 8000
QB = BE // 8

_q = np.arange(ES)
_EDGE_OF_SLOT = ((_q // 8 // QB) * BE + (_q % 8) * QB + (_q // 8) % QB)

_U_SEL = [0, 1, 2, 3, 4, 4, 4, 5, 5, 5, 6, 6, 6, 6, 6]
_S_SEL = [0, 0, 0, 0, 1, 2, 3, 1, 2, 3, 4, 5, 6, 7, 8]


def _sc_gather(nf, src2d):
    mesh = plsc.VectorSubcoreMesh(core_axis_name="c", subcore_axis_name="s")

    @functools.partial(
        pl.kernel,
        mesh=mesh,
        out_type=jax.ShapeDtypeStruct((ES, D_IN), jnp.float32),
        scratch_types=[
            pltpu.VMEM((CH, CHUNK), jnp.int32),
            pltpu.VMEM((CHUNK, D_IN), jnp.float32),
            pltpu.VMEM((CHUNK, D_IN), jnp.float32),
            pltpu.SemaphoreType.DMA,
            pltpu.SemaphoreType.DMA,
        ],
        compiler_params=pltpu.CompilerParams(use_tc_tiling_on_sc=False),
    )
    def k(nf_hbm, src_hbm, out_hbm, idx_v, buf0, buf1, sem0, sem1):
        c = lax.axis_index("c")
        s = lax.axis_index("s")
        wid = s * NC + c
        pltpu.sync_copy(src_hbm.at[pl.ds(wid * CH, CH)], idx_v)
        bufs = (buf0, buf1)
        sems = (sem0, sem1)
        pltpu.async_copy(nf_hbm.at[idx_v.at[0]], buf0, sem0)

        def body(j, carry):
            slot = lax.rem(j, 2)

            def step(b, sm, other_b, other_sm):
                @pl.when(j + 1 < CH)
                def _start():
                    pltpu.async_copy(nf_hbm.at[idx_v.at[j + 1]], other_b, other_sm)

                pltpu.make_async_copy(nf_hbm.at[idx_v.at[j]], b, sm).wait()
                pltpu.sync_copy(b, out_hbm.at[pl.ds(wid * EW + j * CHUNK, CHUNK)])

            @pl.when(slot == 0)
            def _even():
                step(bufs[0], sems[0], bufs[1], sems[1])

            @pl.when(slot == 1)
            def _odd():
                step(bufs[1], sems[1], bufs[0], sems[0])

            return carry

        lax.fori_loop(0, CH, body, 0)

    return k(nf, src2d)


def _sc_scatter(msg, dst2d, zmat):
    mesh = plsc.VectorSubcoreMesh(core_axis_name="c", subcore_axis_name="s")

    @functools.partial(
        pl.kernel,
        mesh=mesh,
        out_type=jax.ShapeDtypeStruct((NC, N_NODES, 16), jnp.float32),
        scratch_types=[
            pltpu.VMEM((CH, CHUNK), jnp.int32),
            pltpu.VMEM((EW, 16), jnp.float32),
            pltpu.VMEM((RPT, 16), jnp.float32),
            pltpu.VMEM_SHARED((N_NODES, 16), jnp.float32),
        ],
        compiler_params=pltpu.CompilerParams(use_tc_tiling_on_sc=False),
    )
    def k(msg_hbm, dst_hbm, z_hbm, out_hbm, idx_v, msg_v, bnc, acc):
        c = lax.axis_index("c")
        s = lax.axis_index("s")
        wid = s * NC + c
        pltpu.sync_copy(z_hbm.at[pl.ds(s * RPT, RPT)], bnc)
        pltpu.sync_copy(bnc, acc.at[pl.ds(s * RPT, RPT)])
        pltpu.sync_copy(dst_hbm.at[pl.ds(wid * CH, CH)], idx_v)
        pltpu.sync_copy(msg_hbm.at[pl.ds(wid * EW, EW)], msg_v)
        plsc.subcore_barrier()

        def body(j, carry):
            pltpu.sync_copy(
                msg_v.at[pl.ds(j * CHUNK, CHUNK)], acc.at[idx_v.at[j]], add=True
            )
            return carry

        lax.fori_loop(0, CH, body, 0)
        plsc.subcore_barrier()
        pltpu.sync_copy(acc.at[pl.ds(s * RPT, RPT)], bnc)
        pltpu.sync_copy(bnc, out_hbm.at[c, pl.ds(s * RPT, RPT)])

    return k(msg, dst2d, zmat)


def _tc_dense(gathered, radial, shp, W1s, W2q, A, B):
    def body(g_ref, r_ref, sh_ref, w1_ref, w2_ref, a_ref, b_ref, o_ref):
        r = r_ref[...]
        h1 = jnp.dot(r, w1_ref[...], preferred_element_type=jnp.float32)
        h = h1 / (1.0 + jnp.exp(-h1))
        w = jnp.dot(h.astype(jnp.bfloat16), w2_ref[...],
                    preferred_element_type=jnp.float32)
        g = g_ref[...]
        cols = []
        for u in range(7):
            cols.append(jnp.sum(w[:, u * D_IN:(u + 1) * D_IN] * g, axis=1,
                                keepdims=True))
        cols.append(jnp.zeros_like(cols[0]))
        proj = jnp.concatenate(cols, axis=1)
        pe = jnp.dot(proj, a_ref[...], preferred_element_type=jnp.float32)
        se = jnp.dot(sh_ref[...], b_ref[...], preferred_element_type=jnp.float32)
        msg = pe * se
        o_ref[...] = jnp.concatenate(
            [msg[k * QB:(k + 1) * QB] for k in range(8)], axis=1)

    return pl.pallas_call(
        body,
        grid=(ES // BE,),
        in_specs=[
            pl.BlockSpec((BE, D_IN), lambda i: (i, 0)),
            pl.BlockSpec((BE, 64), lambda i: (i, 0)),
            pl.BlockSpec((BE, 9), lambda i: (i, 0)),
            pl.BlockSpec((64, 64), lambda i: (0, 0)),
            pl.BlockSpec((64, 896), lambda i: (0, 0)),
            pl.BlockSpec((8, 16), lambda i: (0, 0)),
            pl.BlockSpec((9, 16), lambda i: (0, 0)),
        ],
        out_specs=pl.BlockSpec((QB, 128), lambda i: (i, 0)),
        out_shape=jax.ShapeDtypeStruct((ES // 8, 128), jnp.float32),
    )(gathered, radial, shp, W1s, W2q, A, B)


def _tc_final(partials, nf, wselfp):
    def body(p_ref, nf_ref, ws_ref, o_ref):
        s0 = jnp.dot(nf_ref[...], ws_ref[...], preferred_element_type=jnp.float32)
        acc = s0
        for t in range(NSLICE * NC):
            acc = acc + p_ref[t]
        o_ref[...] = acc

    return pl.pallas_call(
        body,
        out_shape=jax.ShapeDtypeStruct((N_NODES, 16), jnp.float32),
    )(partials, nf, wselfp)


def kernel(node_features, edge_index, edge_sh, edge_radial, W1, W2, W_self):
    src = edge_index[0]
    dst = edge_index[1]
    eq = jnp.asarray(_EDGE_OF_SLOT, dtype=jnp.int32)

    W1s = (W1 * (1.0 / np.sqrt(64.0))).astype(jnp.bfloat16)
    w2_scale = 1.0 / (np.sqrt(64.0) * np.sqrt(float(D_IN)) * 4.0)
    W2q = (W2.reshape(64, D_IN, 7).transpose(0, 2, 1).reshape(64, 7 * D_IN)
           * w2_scale).astype(jnp.bfloat16)

    A = np.zeros((8, 16), np.float32)
    B = np.zeros((16, 16), np.float32)
    for o in range(D_OUT):
        A[_U_SEL[o], o] = 1.0
        B[_S_SEL[o], o] = 1.0
    A = jnp.asarray(A)
    B9 = jnp.asarray(B[:9]).astype(jnp.bfloat16)

    wselfp = jnp.pad(W_self, ((0, 0), (0, 16 - 4))) * (1.0 / np.sqrt(float(D_IN)))
    zmat = jnp.zeros((N_NODES, 16), jnp.float32)

    radial_bf = edge_radial.astype(jnp.bfloat16)
    sh_bf = edge_sh.astype(jnp.bfloat16)

    partials = []
    for sl in range(NSLICE):
        lo = sl * ES
        src2d = src[lo:lo + ES].reshape(ES // CHUNK, CHUNK)
        dst2d = dst[lo:lo + ES][eq].reshape(ES // CHUNK, CHUNK)
        gathered = _sc_gather(node_features, src2d)
        msg8 = _tc_dense(gathered, radial_bf[lo:lo + ES], sh_bf[lo:lo + ES],
                         W1s, W2q, A, B9)
        partials.append(_sc_scatter(msg8.reshape(ES, 16), dst2d, zmat))

    pall = jnp.concatenate(partials, axis=0)
    out16 = _tc_final(pall, node_features, wselfp)
    return out16[:, :D_OUT]
